# baseline (speedup 1.0000x reference)
import functools

import jax
import jax.numpy as jnp
from jax import lax
from jax.experimental import pallas as pl
from jax.experimental.pallas import tpu as pltpu

try:
    from jax.experimental.pallas import tpu_sc as plsc
    _SC_OK = True
except ImportError:
    _SC_OK = False

NE = 8192
ED = 32
TOK = 8192
TILE = 256
CHUNK = 2048
CC = 0.25


def _bf16(x):
    return x.astype(jnp.bfloat16).astype(jnp.float32)


def _vq_body(zb_ref, z2_ref, cb_ref, c2_ref, idx_ref, loss_ref):
    m = jax.lax.dot_general(
        zb_ref[...], cb_ref[...], (((1,), (1,)), ((), ())),
        preferred_element_type=jnp.float32)
    dist = (z2_ref[...] + c2_ref[...]) - m
    lanes = jax.lax.broadcasted_iota(jnp.int32, dist.shape, 1)
    v = None
    for g in range(NE // CHUNK):
        blk = dist[:, g * CHUNK:(g + 1) * CHUNK]
        mg = jnp.min(blk, axis=1, keepdims=True)
        lg = lanes[:, g * CHUNK:(g + 1) * CHUNK]
        ig = jnp.min(jnp.where(blk == mg, lg, jnp.int32(NE)),
                     axis=1, keepdims=True)
        if v is None:
            v, ix, wd = _bf16(mg), ig, mg
        else:
            upd = mg < v
            v = jnp.where(upd, _bf16(mg), v)
            ix = jnp.where(upd, ig, ix)
            wd = jnp.where(upd, mg, wd)
    idx_ref[...] = ix

    @pl.when(pl.program_id(0) == 0)
    def _init():
        loss_ref[0, 0] = 0.0

    loss_ref[0, 0] += jnp.sum(wd)


def _vq_call(zb, z2, cb, c2):
    grid = TOK // TILE
    return pl.pallas_call(
        _vq_body,
        grid=(grid,),
        in_specs=[
            pl.BlockSpec((TILE, ED), lambda i: (i, 0)),
            pl.BlockSpec((TILE, 1), lambda i: (i, 0)),
            pl.BlockSpec((NE, ED), lambda i: (0, 0)),
            pl.BlockSpec((1, NE), lambda i: (0, 0)),
        ],
        out_specs=[
            pl.BlockSpec((TILE, 1), lambda i: (i, 0)),
            pl.BlockSpec(memory_space=pltpu.SMEM),
        ],
        out_shape=[
            jax.ShapeDtypeStruct((TOK, 1), jnp.int32),
            jax.ShapeDtypeStruct((1, 1), jnp.float32),
        ],
    )(zb, z2, cb, c2)


def _make_sc_gather():
    info = plsc.get_sparse_core_info()
    nw = info.num_cores * info.num_subcores
    n_per_w = TOK * ED // nw
    mesh = plsc.VectorSubcoreMesh(core_axis_name="c", subcore_axis_name="s")

    @functools.partial(
        pl.kernel, mesh=mesh,
        out_type=jax.ShapeDtypeStruct((TOK * ED,), jnp.float32),
        scratch_types=[
            pltpu.VMEM((n_per_w,), jnp.int32),
            pltpu.VMEM((n_per_w,), jnp.float32),
            pltpu.SemaphoreType.DMA,
        ],
    )
    def sc_gather(table_hbm, idx_hbm, out_hbm, idx_v, vals_v, sem):
        wid = lax.axis_index("s") * info.num_cores + lax.axis_index("c")
        base = wid * n_per_w
        pltpu.sync_copy(idx_hbm.at[pl.ds(base, n_per_w)], idx_v)
        pltpu.async_copy(table_hbm.at[idx_v], vals_v, sem).wait()
        pltpu.sync_copy(vals_v, out_hbm.at[pl.ds(base, n_per_w)])

    return sc_gather


def kernel(z, codebook):
    zp = jnp.transpose(z, (0, 2, 1))
    z_flat = zp.reshape(-1, ED)
    z2 = jnp.sum(z_flat ** 2, axis=1, keepdims=True)
    c2 = jnp.sum(codebook ** 2, axis=1).reshape(1, NE)
    zb = (2.0 * z_flat).astype(jnp.bfloat16)
    cb = codebook.astype(jnp.bfloat16)
    idx2d, loss_sum = _vq_call(zb, z2, cb, c2)
    vq_loss = loss_sum[0, 0] * ((1.0 + CC) / (TOK * ED))
    table = cb.astype(jnp.float32).reshape(-1)
    idx32 = (idx2d * ED + jnp.arange(ED, dtype=jnp.int32)[None, :]).reshape(-1)
    q_flat = _make_sc_gather()(table, idx32).reshape(TOK, ED)
    q_st = z_flat + (q_flat - z_flat)
    q_out = jnp.transpose(q_st.reshape(zp.shape), (0, 2, 1))
    idx_out = idx2d.reshape(z.shape[0], -1)
    return (vq_loss, q_out, idx_out)


# speedup vs baseline: 1.0990x; 1.0990x over previous
import jax
import jax.numpy as jnp
from jax.experimental import pallas as pl
from jax.experimental.pallas import tpu as pltpu

NE = 8192
ED = 32
TOK = 8192
TILE = 256
CHUNK = 2048
CC = 0.25


def _bf16(x):
    return x.astype(jnp.bfloat16).astype(jnp.float32)


def _vq_body(zb_ref, z2_ref, cb_ref, c2_ref, zf_ref, idx_ref, q_ref, loss_ref):
    m = jax.lax.dot_general(
        zb_ref[...], cb_ref[...], (((1,), (1,)), ((), ())),
        preferred_element_type=jnp.float32)
    dist = (z2_ref[...] + c2_ref[...]) - m
    lanes = jax.lax.broadcasted_iota(jnp.int32, dist.shape, 1)
    v = None
    for g in range(NE // CHUNK):
        blk = dist[:, g * CHUNK:(g + 1) * CHUNK]
        mg = jnp.min(blk, axis=1, keepdims=True)
        lg = lanes[:, g * CHUNK:(g + 1) * CHUNK]
        ig = jnp.min(jnp.where(blk == mg, lg, jnp.int32(NE)),
                     axis=1, keepdims=True)
        if v is None:
            v, ix, wd = _bf16(mg), ig, mg
        else:
            upd = mg < v
            v = jnp.where(upd, _bf16(mg), v)
            ix = jnp.where(upd, ig, ix)
            wd = jnp.where(upd, mg, wd)
    idx_ref[...] = ix
    enc = jnp.where(lanes == ix, 1.0, 0.0).astype(jnp.bfloat16)
    q = jax.lax.dot_general(
        enc, cb_ref[...], (((1,), (0,)), ((), ())),
        preferred_element_type=jnp.float32)
    zf = zf_ref[...]
    q_ref[...] = zf + (q - zf)

    @pl.when(pl.program_id(0) == 0)
    def _init():
        loss_ref[0, 0] = 0.0

    loss_ref[0, 0] += jnp.sum(wd)


def _vq_call(zb, z2, cb, c2, zf):
    grid = TOK // TILE
    return pl.pallas_call(
        _vq_body,
        grid=(grid,),
        in_specs=[
            pl.BlockSpec((TILE, ED), lambda i: (i, 0)),
            pl.BlockSpec((TILE, 1), lambda i: (i, 0)),
            pl.BlockSpec((NE, ED), lambda i: (0, 0)),
            pl.BlockSpec((1, NE), lambda i: (0, 0)),
            pl.BlockSpec((TILE, ED), lambda i: (i, 0)),
        ],
        out_specs=[
            pl.BlockSpec((TILE, 1), lambda i: (i, 0)),
            pl.BlockSpec((TILE, ED), lambda i: (i, 0)),
            pl.BlockSpec(memory_space=pltpu.SMEM),
        ],
        out_shape=[
            jax.ShapeDtypeStruct((TOK, 1), jnp.int32),
            jax.ShapeDtypeStruct((TOK, ED), jnp.float32),
            jax.ShapeDtypeStruct((1, 1), jnp.float32),
        ],
    )(zb, z2, cb, c2, zf)


def kernel(z, codebook):
    zp = jnp.transpose(z, (0, 2, 1))
    z_flat = zp.reshape(-1, ED)
    z2 = jnp.sum(z_flat ** 2, axis=1, keepdims=True)
    c2 = jnp.sum(codebook ** 2, axis=1).reshape(1, NE)
    zb = (2.0 * z_flat).astype(jnp.bfloat16)
    cb = codebook.astype(jnp.bfloat16)
    idx2d, q_st, loss_sum = _vq_call(zb, z2, cb, c2, z_flat)
    vq_loss = loss_sum[0, 0] * ((1.0 + CC) / (TOK * ED))
    q_out = jnp.transpose(q_st.reshape(zp.shape), (0, 2, 1))
    idx_out = idx2d.reshape(z.shape[0], -1)
    return (vq_loss, q_out, idx_out)


# speedup vs baseline: 1.2318x; 1.1209x over previous
import jax
import jax.numpy as jnp
from jax.experimental import pallas as pl
from jax.experimental.pallas import tpu as pltpu

NE = 8192
ED = 32
TOK = 8192
TILE = 256
CHUNK = 2048
CC = 0.25


def _bf16(x):
    return x.astype(jnp.bfloat16).astype(jnp.float32)


def _vq_body(zb_ref, z2_ref, cb_ref, c2_ref, zf_ref, idx_ref, q_ref, loss_ref):
    m = jax.lax.dot_general(
        zb_ref[...], cb_ref[...], (((1,), (1,)), ((), ())),
        preferred_element_type=jnp.float32)
    dist = (z2_ref[...] + c2_ref[...]) - m
    lanes = jax.lax.broadcasted_iota(jnp.int32, dist.shape, 1).astype(jnp.float32)
    v = None
    for g in range(NE // CHUNK):
        blk = dist[:, g * CHUNK:(g + 1) * CHUNK]
        mg = jnp.min(blk, axis=1, keepdims=True)
        lg = lanes[:, g * CHUNK:(g + 1) * CHUNK]
        ig = jnp.min(jnp.where(blk == mg, lg, jnp.float32(NE)),
                     axis=1, keepdims=True)
        if v is None:
            v, ix, wd = _bf16(mg), ig, mg
        else:
            upd = mg < v
            v = jnp.where(upd, _bf16(mg), v)
            ix = jnp.where(upd, ig, ix)
            wd = jnp.where(upd, mg, wd)
    idx_ref[...] = ix.astype(jnp.int32)
    enc = jnp.where(lanes == ix, 1.0, 0.0).astype(jnp.bfloat16)
    q = jax.lax.dot_general(
        enc, cb_ref[...], (((1,), (0,)), ((), ())),
        preferred_element_type=jnp.float32)
    zf = zf_ref[...]
    q_ref[...] = zf + (q - zf)

    @pl.when(pl.program_id(0) == 0)
    def _init():
        loss_ref[0, 0] = 0.0

    loss_ref[0, 0] += jnp.sum(wd)


def _vq_call(zb, z2, cb, c2, zf):
    grid = TOK // TILE
    return pl.pallas_call(
        _vq_body,
        grid=(grid,),
        in_specs=[
            pl.BlockSpec((TILE, ED), lambda i: (i, 0)),
            pl.BlockSpec((TILE, 1), lambda i: (i, 0)),
            pl.BlockSpec((NE, ED), lambda i: (0, 0)),
            pl.BlockSpec((1, NE), lambda i: (0, 0)),
            pl.BlockSpec((TILE, ED), lambda i: (i, 0)),
        ],
        out_specs=[
            pl.BlockSpec((TILE, 1), lambda i: (i, 0)),
            pl.BlockSpec((TILE, ED), lambda i: (i, 0)),
            pl.BlockSpec(memory_space=pltpu.SMEM),
        ],
        out_shape=[
            jax.ShapeDtypeStruct((TOK, 1), jnp.int32),
            jax.ShapeDtypeStruct((TOK, ED), jnp.float32),
            jax.ShapeDtypeStruct((1, 1), jnp.float32),
        ],
    )(zb, z2, cb, c2, zf)


def kernel(z, codebook):
    zp = jnp.transpose(z, (0, 2, 1))
    z_flat = zp.reshape(-1, ED)
    z2 = jnp.sum(z_flat ** 2, axis=1, keepdims=True)
    c2 = jnp.sum(codebook ** 2, axis=1).reshape(1, NE)
    zb = (2.0 * z_flat).astype(jnp.bfloat16)
    cb = codebook.astype(jnp.bfloat16)
    idx2d, q_st, loss_sum = _vq_call(zb, z2, cb, c2, z_flat)
    vq_loss = loss_sum[0, 0] * ((1.0 + CC) / (TOK * ED))
    q_out = jnp.transpose(q_st.reshape(zp.shape), (0, 2, 1))
    idx_out = idx2d.reshape(z.shape[0], -1)
    return (vq_loss, q_out, idx_out)


# speedup vs baseline: 1.2723x; 1.0328x over previous
import jax
import jax.numpy as jnp
from jax.experimental import pallas as pl
from jax.experimental.pallas import tpu as pltpu

NE = 8192
ED = 32
TOK = 8192
TILE = 256
LSUB = 1024 // TILE
CHUNK = 2048
CC = 0.25


def _bf16(x):
    return x.astype(jnp.bfloat16).astype(jnp.float32)


def _vq_body(z_ref, z2_ref, cb_ref, c2_ref, idx_ref, q_ref, loss_ref):
    zt = z_ref[0]
    zf = jnp.transpose(zt, (1, 0))
    zb = (2.0 * zf).astype(jnp.bfloat16)
    m = jax.lax.dot_general(
        zb, cb_ref[...], (((1,), (1,)), ((), ())),
        preferred_element_type=jnp.float32)
    dist = (z2_ref[...] + c2_ref[...]) - m
    lanes = jax.lax.broadcasted_iota(jnp.int32, dist.shape, 1).astype(jnp.float32)
    v = None
    for g in range(NE // CHUNK):
        blk = dist[:, g * CHUNK:(g + 1) * CHUNK]
        mg = jnp.min(blk, axis=1, keepdims=True)
        lg = lanes[:, g * CHUNK:(g + 1) * CHUNK]
        ig = jnp.min(jnp.where(blk == mg, lg, jnp.float32(NE)),
                     axis=1, keepdims=True)
        if v is None:
            v, ix, wd = _bf16(mg), ig, mg
        else:
            upd = mg < v
            v = jnp.where(upd, _bf16(mg), v)
            ix = jnp.where(upd, ig, ix)
            wd = jnp.where(upd, mg, wd)
    idx_ref[...] = ix.astype(jnp.int32)
    enc = jnp.where(lanes == ix, 1.0, 0.0).astype(jnp.bfloat16)
    q = jax.lax.dot_general(
        enc, cb_ref[...], (((1,), (0,)), ((), ())),
        preferred_element_type=jnp.float32)
    q_st = zf + (q - zf)
    q_ref[0] = jnp.transpose(q_st, (1, 0))

    @pl.when(pl.program_id(0) == 0)
    def _init():
        loss_ref[0, 0] = 0.0

    loss_ref[0, 0] += jnp.sum(wd)


def _vq_call(z, z2, cb, c2):
    grid = TOK // TILE
    return pl.pallas_call(
        _vq_body,
        grid=(grid,),
        in_specs=[
            pl.BlockSpec((1, ED, TILE), lambda i: (i // LSUB, 0, i % LSUB)),
            pl.BlockSpec((TILE, 1), lambda i: (i, 0)),
            pl.BlockSpec((NE, ED), lambda i: (0, 0)),
            pl.BlockSpec((1, NE), lambda i: (0, 0)),
        ],
        out_specs=[
            pl.BlockSpec((TILE, 1), lambda i: (i, 0)),
            pl.BlockSpec((1, ED, TILE), lambda i: (i // LSUB, 0, i % LSUB)),
            pl.BlockSpec(memory_space=pltpu.SMEM),
        ],
        out_shape=[
            jax.ShapeDtypeStruct((TOK, 1), jnp.int32),
            jax.ShapeDtypeStruct((8, ED, 1024), jnp.float32),
            jax.ShapeDtypeStruct((1, 1), jnp.float32),
        ],
    )(z, z2, cb, c2)


def kernel(z, codebook):
    zp = jnp.transpose(z, (0, 2, 1))
    z_flat = zp.reshape(-1, ED)
    z2 = jnp.sum(z_flat ** 2, axis=1, keepdims=True)
    c2 = jnp.sum(codebook ** 2, axis=1).reshape(1, NE)
    cb = codebook.astype(jnp.bfloat16)
    idx2d, q_out, loss_sum = _vq_call(z, z2, cb, c2)
    vq_loss = loss_sum[0, 0] * ((1.0 + CC) / (TOK * ED))
    idx_out = idx2d.reshape(z.shape[0], -1)
    return (vq_loss, q_out, idx_out)
